# baseline (speedup 1.0000x reference)
import jax
import jax.numpy as jnp
from jax.experimental import pallas as pl
from jax.experimental.pallas import tpu as pltpu

_KC = 40


def _qtab_kernel(in_ref, mask_ref, table_ref, out_ref):
    k = pl.program_id(0)
    acc = jnp.dot(
        table_ref[0].astype(jnp.bfloat16),
        in_ref[0].astype(jnp.bfloat16),
        preferred_element_type=jnp.float32,
    )

    @pl.when(k == 0)
    def _first():
        out_ref[...] = acc + mask_ref[...]

    @pl.when(k > 0)
    def _rest():
        out_ref[...] = out_ref[...] + acc


def kernel(inputs, mask, table):
    B, K = inputs.shape
    N = table.shape[1]
    nk = K // _KC
    table_chunks = table.T.reshape(N, nk, _KC).swapaxes(0, 1)
    in_chunks = inputs.T.reshape(nk, _KC, B)
    out_t = pl.pallas_call(
        _qtab_kernel,
        grid=(nk,),
        in_specs=[
            pl.BlockSpec((1, _KC, B), lambda k: (k, 0, 0)),
            pl.BlockSpec((N, B), lambda k: (0, 0)),
            pl.BlockSpec((1, N, _KC), lambda k: (k, 0, 0)),
        ],
        out_specs=pl.BlockSpec((N, B), lambda k: (0, 0)),
        out_shape=jax.ShapeDtypeStruct((N, B), jnp.float32),
        compiler_params=pltpu.CompilerParams(
            dimension_semantics=("arbitrary",),
        ),
    )(in_chunks, mask.T, table_chunks)
    return out_t.T


# speedup vs baseline: 1.2947x; 1.2947x over previous
import jax
import jax.numpy as jnp
from jax.experimental import pallas as pl
from jax.experimental.pallas import tpu as pltpu

_KC = 40
_DEPTH = 8


def _qtab_kernel(in_hbm, mask_ref, table_ref, out_ref, bufs, sems):
    nk = in_hbm.shape[0]

    def start(chunk, slot):
        pltpu.make_async_copy(
            in_hbm.at[chunk], bufs.at[slot], sems.at[slot]
        ).start(priority=slot % 2)

    for slot in range(min(_DEPTH, nk)):
        start(slot, slot)
    for i in range(nk):
        slot = i % _DEPTH
        pltpu.make_async_copy(
            in_hbm.at[i], bufs.at[slot], sems.at[slot]
        ).wait()
        acc = jnp.dot(
            table_ref[i].astype(jnp.bfloat16),
            bufs[slot].astype(jnp.bfloat16),
            preferred_element_type=jnp.float32,
        )
        if i == 0:
            out_ref[...] = acc + mask_ref[...]
        else:
            out_ref[...] = out_ref[...] + acc
        nxt = i + _DEPTH
        if nxt < nk:
            start(nxt, slot)


def kernel(inputs, mask, table):
    B, K = inputs.shape
    N = table.shape[1]
    nk = K // _KC
    in_chunks = inputs.T.reshape(nk, _KC, B)
    table_chunks = table.T.reshape(N, nk, _KC).swapaxes(0, 1)
    out_t = pl.pallas_call(
        _qtab_kernel,
        in_specs=[
            pl.BlockSpec(memory_space=pltpu.MemorySpace.HBM),
            pl.BlockSpec(memory_space=pltpu.MemorySpace.VMEM),
            pl.BlockSpec(memory_space=pltpu.MemorySpace.VMEM),
        ],
        out_specs=pl.BlockSpec(memory_space=pltpu.MemorySpace.VMEM),
        out_shape=jax.ShapeDtypeStruct((N, B), jnp.float32),
        scratch_shapes=[
            pltpu.VMEM((_DEPTH, _KC, B), jnp.float32),
            pltpu.SemaphoreType.DMA((_DEPTH,)),
        ],
    )(in_chunks, mask.T, table_chunks)
    return out_t.T


# speedup vs baseline: 1.5174x; 1.1720x over previous
import jax
import jax.numpy as jnp
from jax.experimental import pallas as pl
from jax.experimental.pallas import tpu as pltpu

_BN = 2048


def _qtab_kernel(in_ref, mask_ref, table_ref, out_ref):
    i = pl.program_id(0)
    a = table_ref[...].astype(jnp.bfloat16)
    b = in_ref[...].astype(jnp.bfloat16)
    out_ref[:, pl.ds(i * _BN, _BN)] = (
        jnp.dot(a, b, preferred_element_type=jnp.float32)
        + mask_ref[:, pl.ds(i * _BN, _BN)]
    )


def kernel(inputs, mask, table):
    B, K = inputs.shape
    N = table.shape[1]
    out_t = pl.pallas_call(
        _qtab_kernel,
        grid=(B // _BN,),
        in_specs=[
            pl.BlockSpec((K, _BN), lambda i: (0, i)),
            pl.BlockSpec((N, B), lambda i: (0, 0)),
            pl.BlockSpec((N, K), lambda i: (0, 0)),
        ],
        out_specs=pl.BlockSpec((N, B), lambda i: (0, 0)),
        out_shape=jax.ShapeDtypeStruct((N, B), jnp.float32),
        compiler_params=pltpu.CompilerParams(
            dimension_semantics=("arbitrary",),
        ),
    )(inputs.T, mask.T, table.T)
    return out_t.T
